# baseline (speedup 1.0000x reference)
import functools

import jax
import jax.numpy as jnp
from jax import lax
from jax.experimental import pallas as pl
from jax.experimental.pallas import tpu as pltpu
from jax.experimental.pallas import tpu_sc as plsc

D = 64
DP = 128
NC = 2
NS = 16
LANES = 16
CHUNK = 128
NBUF = 2
NOBUF = 2


@functools.lru_cache(maxsize=None)
def _build(B, L):
    N = B * L
    NW = NC * NS
    per_w = N // NW
    nch = per_w // CHUNK
    lt_per_b = L // CHUNK
    rows_w = per_w // L

    mesh = plsc.VectorSubcoreMesh(core_axis_name="c", subcore_axis_name="s")

    @functools.partial(
        pl.kernel,
        mesh=mesh,
        out_type=jax.ShapeDtypeStruct((B, D, L), jnp.float32),
        compiler_params=pltpu.CompilerParams(
            use_tc_tiling_on_sc=True, needs_layout_passes=False),
        scratch_types=[pltpu.VMEM((per_w,), jnp.int32),
                       pltpu.VMEM((lt_per_b, D, CHUNK), jnp.float32)]
                      + [pltpu.VMEM((CHUNK, DP), jnp.float32) for _ in range(NBUF)]
                      + [pltpu.VMEM((D, CHUNK), jnp.float32) for _ in range(NOBUF)]
                      + [pltpu.SemaphoreType.DMA for _ in range(NBUF)]
                      + [pltpu.SemaphoreType.DMA for _ in range(NOBUF)],
    )
    def k(x_hbm, tok_hbm, pos_hbm, out_hbm, idx_v, pos_v, *rest):
        gbufs = rest[:NBUF]
        obufs = rest[NBUF:NBUF + NOBUF]
        gsems = rest[NBUF + NOBUF:NBUF + NOBUF + NBUF]
        osems = rest[NBUF + NOBUF + NBUF:]
        wid = lax.axis_index("s") * NC + lax.axis_index("c")
        base = wid * per_w
        brow0 = wid * rows_w

        pltpu.sync_copy(x_hbm.at[pl.ds(base, per_w)], idx_v)
        for lt in range(lt_per_b):
            pltpu.sync_copy(pos_hbm.at[:, pl.ds(lt * CHUNK, CHUNK)],
                            pos_v.at[lt])

        def gather_start(c, gb):
            pltpu.async_copy(
                tok_hbm.at[idx_v.at[pl.ds(c * CHUNK, CHUNK)]], gbufs[gb], gsems[gb])

        def gather_wait(gb):
            pltpu.make_async_copy(
                tok_hbm.at[pl.ds(0, CHUNK)], gbufs[gb], gsems[gb]).wait()

        def out_start(brow, lt, ob):
            pltpu.async_copy(
                obufs[ob],
                out_hbm.at[brow, :, pl.ds(lt * CHUNK, CHUNK)],
                osems[ob])

        def out_wait(ob):
            pltpu.make_async_copy(
                out_hbm.at[0, :, pl.ds(0, CHUNK)], obufs[ob], osems[ob]).wait()

        def compute(lt, gb, ob):
            def drow(d, carry):
                cols = jnp.full((LANES,), d, jnp.int32)
                for kk in range(CHUNK // LANES):
                    rows = lax.iota(jnp.int32, LANES) + (kk * LANES)
                    g = plsc.load_gather(gbufs[gb], [rows, cols])
                    p = pos_v[lt, d, pl.ds(kk * LANES, LANES)]
                    obufs[ob][d, pl.ds(kk * LANES, LANES)] = g + p
                return carry
            lax.fori_loop(0, D, drow, 0)

        for gb in range(NBUF):
            gather_start(gb, gb)

        def do_chunk(c, lt, j, start_next, wait_out):
            gather_wait(j % NBUF)
            if wait_out:
                out_wait(j % NOBUF)
            compute(lt, j % NBUF, j % NOBUF)
            out_start(base // L + c // lt_per_b, lt, j % NOBUF)
            if start_next:
                gather_start(c + NBUF, j % NBUF)

        for j in range(lt_per_b):
            do_chunk(j, j, j, True, j >= NOBUF)

        def group(g, carry):
            for j in range(lt_per_b):
                do_chunk(g * lt_per_b + j, j, j, True, True)
            return carry
        lax.fori_loop(1, rows_w - 1, group, 0)
        for j in range(lt_per_b):
            do_chunk(nch - lt_per_b + j, j, j, j < lt_per_b - NBUF, True)
        for ob in range(NOBUF):
            out_wait(ob)

    return k


def kernel(x, token_table, pos_table):
    B, L = x.shape
    xf = x.reshape(B * L).astype(jnp.int32)
    tok_p = jnp.pad(token_table, ((0, 0), (0, DP - D)))
    pos_t = pos_table.T
    out3 = _build(B, L)(xf, tok_p, pos_t)
    return out3.swapaxes(1, 2)


# speedup vs baseline: 1.1935x; 1.1935x over previous
import functools

import jax
import jax.numpy as jnp
from jax import lax
from jax.experimental import pallas as pl
from jax.experimental.pallas import tpu as pltpu
from jax.experimental.pallas import tpu_sc as plsc

D = 64
DP = 128
NC = 2
NS = 16
LANES = 16
CHUNK = 128
NBUF = 2
V_BLK = 2048


@functools.lru_cache(maxsize=None)
def _build_transpose(V):
    grid = pl.cdiv(V, V_BLK)

    def body(in_ref, o_ref):
        o_ref[:, :D] = in_ref[...].T
        o_ref[:, D:] = jnp.zeros((V_BLK, DP - D), jnp.float32)

    return pl.pallas_call(
        body,
        grid=(grid,),
        in_specs=[pl.BlockSpec((D, V_BLK), lambda i: (0, i))],
        out_specs=pl.BlockSpec((V_BLK, DP), lambda i: (i, 0)),
        out_shape=jax.ShapeDtypeStruct((V, DP), jnp.float32),
    )


@functools.lru_cache(maxsize=None)
def _build_lookup(N, V, L_POS):
    NW = NC * NS
    per_w = N // NW
    nch = per_w // CHUNK

    mesh = plsc.VectorSubcoreMesh(core_axis_name="c", subcore_axis_name="s")

    @functools.partial(
        pl.kernel,
        mesh=mesh,
        out_type=jax.ShapeDtypeStruct((N, DP), jnp.float32),
        compiler_params=pltpu.CompilerParams(
            use_tc_tiling_on_sc=True, needs_layout_passes=False),
        scratch_types=[pltpu.VMEM((per_w,), jnp.int32),
                       pltpu.VMEM((L_POS, DP), jnp.float32)]
                      + [pltpu.VMEM((CHUNK, DP), jnp.float32) for _ in range(NBUF)]
                      + [pltpu.SemaphoreType.DMA for _ in range(NBUF)]
                      + [pltpu.SemaphoreType.DMA],
    )
    def k(x_hbm, tok_hbm, pos_hbm, out_hbm, idx_v, pos_v, *rest):
        bufs = rest[:NBUF]
        sems = rest[NBUF:NBUF + NBUF]
        osem = rest[NBUF + NBUF]
        wid = lax.axis_index("s") * NC + lax.axis_index("c")
        base = wid * per_w

        pltpu.sync_copy(x_hbm.at[pl.ds(base, per_w)], idx_v)
        pltpu.sync_copy(pos_hbm, pos_v)

        def gather_start(c, b):
            pltpu.async_copy(
                tok_hbm.at[idx_v.at[pl.ds(c * CHUNK, CHUNK)]], bufs[b], sems[b])

        def gather_wait(b):
            pltpu.make_async_copy(
                tok_hbm.at[pl.ds(0, CHUNK)], bufs[b], sems[b]).wait()

        def add_pos(b, c):
            pbase = (c * CHUNK) % L_POS

            def row(r, carry):
                pr = pbase + r
                for j in range(DP // LANES):
                    plsc.addupdate(bufs[b].at[r, pl.ds(j * LANES, LANES)],
                                   pos_v[pr, pl.ds(j * LANES, LANES)])
                return carry
            lax.fori_loop(0, CHUNK, row, 0)

        def do_chunk(c, b, start_next):
            gather_wait(b)
            add_pos(b, c)
            pltpu.sync_copy(bufs[b],
                            out_hbm.at[pl.ds(base + c * CHUNK, CHUNK)])
            if start_next:
                gather_start(c + NBUF, b)

        for b in range(NBUF):
            gather_start(b, b)

        def group(g, carry):
            for b in range(NBUF):
                do_chunk(g * NBUF + b, b, True)
            return carry
        lax.fori_loop(0, nch // NBUF - 1, group, 0)
        for b in range(NBUF):
            do_chunk(nch - NBUF + b, b, False)

    return k


def kernel(x, token_table, pos_table):
    B, L = x.shape
    V = token_table.shape[0]
    xf = x.reshape(B * L).astype(jnp.int32)
    tok_p = _build_transpose(V)(token_table.T)
    pos_p = jnp.pad(pos_table, ((0, 0), (0, DP - D)))
    out = _build_lookup(B * L, V, L)(xf, tok_p, pos_p)
    return out[:, :D].reshape(B, L, D)


# speedup vs baseline: 1.6553x; 1.3869x over previous
import functools

import jax
import jax.numpy as jnp
from jax import lax
from jax.experimental import pallas as pl
from jax.experimental.pallas import tpu as pltpu
from jax.experimental.pallas import tpu_sc as plsc

D = 64
DP = 128
NC = 2
NS = 16
LANES = 16
CHUNK = 64
NBUF = 4
V_BLK = 2048


@functools.lru_cache(maxsize=None)
def _build_transpose(V):
    grid = pl.cdiv(V, V_BLK)

    def body(in_ref, o_ref):
        o_ref[:, :D] = in_ref[...].T
        o_ref[:, D:] = jnp.zeros((V_BLK, DP - D), jnp.float32)

    return pl.pallas_call(
        body,
        grid=(grid,),
        in_specs=[pl.BlockSpec((D, V_BLK), lambda i: (0, i))],
        out_specs=pl.BlockSpec((V_BLK, DP), lambda i: (i, 0)),
        out_shape=jax.ShapeDtypeStruct((V, DP), jnp.float32),
    )


@functools.lru_cache(maxsize=None)
def _build_lookup(N, V, L_POS):
    NW = NC * NS
    per_w = N // NW
    nch = per_w // CHUNK

    mesh = plsc.VectorSubcoreMesh(core_axis_name="c", subcore_axis_name="s")

    @functools.partial(
        pl.kernel,
        mesh=mesh,
        out_type=jax.ShapeDtypeStruct((N, DP), jnp.float32),
        compiler_params=pltpu.CompilerParams(
            use_tc_tiling_on_sc=True, needs_layout_passes=False),
        scratch_types=[pltpu.VMEM((per_w,), jnp.int32),
                       pltpu.VMEM((L_POS, DP), jnp.float32)]
                      + [pltpu.VMEM((CHUNK, DP), jnp.float32) for _ in range(NBUF)]
                      + [pltpu.SemaphoreType.DMA for _ in range(NBUF)]
                      + [pltpu.SemaphoreType.DMA for _ in range(NBUF)],
    )
    def k(x_hbm, tok_hbm, pos_hbm, out_hbm, idx_v, pos_v, *rest):
        bufs = rest[:NBUF]
        sems = rest[NBUF:NBUF + NBUF]
        osems = rest[NBUF + NBUF:]
        wid = lax.axis_index("s") * NC + lax.axis_index("c")
        base = wid * per_w

        pltpu.sync_copy(x_hbm.at[pl.ds(base, per_w)], idx_v)
        pltpu.sync_copy(pos_hbm, pos_v)

        def gather_start(c, b):
            pltpu.async_copy(
                tok_hbm.at[idx_v.at[pl.ds(c * CHUNK, CHUNK)]], bufs[b], sems[b])

        def gather_wait(b):
            pltpu.make_async_copy(
                tok_hbm.at[pl.ds(0, CHUNK)], bufs[b], sems[b]).wait()

        def add_pos(b, c):
            pbase = (c * CHUNK) % L_POS

            def row(r, carry):
                pr = pbase + r
                for j in range(D // LANES):
                    plsc.addupdate(bufs[b].at[r, pl.ds(j * LANES, LANES)],
                                   pos_v[pr, pl.ds(j * LANES, LANES)])
                return carry
            lax.fori_loop(0, CHUNK, row, 0)

        def out_start(c, b):
            pltpu.async_copy(bufs[b],
                             out_hbm.at[pl.ds(base + c * CHUNK, CHUNK)],
                             osems[b])

        def out_wait(b):
            pltpu.make_async_copy(
                out_hbm.at[pl.ds(0, CHUNK)], bufs[b], osems[b]).wait()

        def do_chunk(c, b, handle_prev, start_prev_next):
            gather_wait(b)
            add_pos(b, c)
            out_start(c, b)
            if handle_prev:
                pb = (b - 1) % NBUF
                out_wait(pb)
                if start_prev_next:
                    gather_start(c - 1 + NBUF, pb)

        for b in range(NBUF):
            gather_start(b, b)

        for b in range(NBUF):
            do_chunk(b, b, b >= 1, True)

        def group(g, carry):
            c0 = g * NBUF
            for b in range(NBUF):
                do_chunk(c0 + b, b, True, True)
            return carry
        lax.fori_loop(1, nch // NBUF - 1, group, 0)
        for b in range(NBUF):
            c = nch - NBUF + b
            do_chunk(c, b, True, c - 1 + NBUF < nch)
        out_wait((nch - 1) % NBUF)

    return k


def kernel(x, token_table, pos_table):
    B, L = x.shape
    V = token_table.shape[0]
    xf = x.reshape(B * L).astype(jnp.int32)
    tok_p = _build_transpose(V)(token_table.T)
    pos_p = jnp.pad(pos_table, ((0, 0), (0, DP - D)))
    out = _build_lookup(B * L, V, L)(xf, tok_p, pos_p)
    return out[:, :D].reshape(B, L, D)


# speedup vs baseline: 2.0390x; 1.2318x over previous
import functools

import jax
import jax.numpy as jnp
from jax import lax
from jax.experimental import pallas as pl
from jax.experimental.pallas import tpu as pltpu
from jax.experimental.pallas import tpu_sc as plsc

D = 64
DP = 128
NC = 2
NS = 16
LANES = 16
CHUNK = 128
NBUF = 4
V_BLK = 4096


@functools.lru_cache(maxsize=None)
def _build_transpose(V):
    grid = pl.cdiv(V, V_BLK)

    def body(in_ref, o_ref):
        o_ref[:, :D] = in_ref[...].T

    return pl.pallas_call(
        body,
        grid=(grid,),
        in_specs=[pl.BlockSpec((D, V_BLK), lambda i: (0, i))],
        out_specs=pl.BlockSpec((V_BLK, DP), lambda i: (i, 0)),
        out_shape=jax.ShapeDtypeStruct((V, DP), jnp.float32),
    )


@functools.lru_cache(maxsize=None)
def _build_lookup(N, V, L_POS):
    NW = NC * NS
    per_w = N // NW
    nch = per_w // CHUNK

    mesh = plsc.VectorSubcoreMesh(core_axis_name="c", subcore_axis_name="s")

    @functools.partial(
        pl.kernel,
        mesh=mesh,
        out_type=jax.ShapeDtypeStruct((N, DP), jnp.float32),
        compiler_params=pltpu.CompilerParams(
            use_tc_tiling_on_sc=True, needs_layout_passes=False),
        scratch_types=[pltpu.VMEM((per_w,), jnp.int32),
                       pltpu.VMEM((L_POS // 2, DP), jnp.float32)]
                      + [pltpu.VMEM((CHUNK, DP), jnp.float32) for _ in range(NBUF)]
                      + [pltpu.SemaphoreType.DMA for _ in range(NBUF)]
                      + [pltpu.SemaphoreType.DMA for _ in range(NBUF)],
    )
    def k(x_hbm, tok_hbm, pos_hbm, out_hbm, idx_v, pos_v, *rest):
        bufs = rest[:NBUF]
        sems = rest[NBUF:NBUF + NBUF]
        osems = rest[NBUF + NBUF:]
        wid = lax.axis_index("s") * NC + lax.axis_index("c")
        base = wid * per_w

        pltpu.sync_copy(x_hbm.at[pl.ds(base, per_w)], idx_v)
        pltpu.sync_copy(pos_hbm, pos_v)

        def gather_start(c, b):
            pltpu.async_copy(
                tok_hbm.at[idx_v.at[pl.ds(c * CHUNK, CHUNK)]], bufs[b], sems[b])

        def gather_wait(b):
            pltpu.make_async_copy(
                tok_hbm.at[pl.ds(0, CHUNK)], bufs[b], sems[b]).wait()

        def add_pos(b, c):
            pbase = (c * CHUNK) % L_POS

            def row(r, carry):
                pr = pbase + r
                ph = pr // 2
                po = (pr % 2) * D
                for j in range(D // LANES):
                    plsc.addupdate(bufs[b].at[r, pl.ds(j * LANES, LANES)],
                                   pos_v[ph, pl.ds(po + j * LANES, LANES)])
                return carry
            lax.fori_loop(0, CHUNK, row, 0)

        def out_start(c, b):
            pltpu.async_copy(bufs[b],
                             out_hbm.at[pl.ds(base + c * CHUNK, CHUNK)],
                             osems[b])

        def out_wait(b):
            pltpu.make_async_copy(
                out_hbm.at[pl.ds(0, CHUNK)], bufs[b], osems[b]).wait()

        def do_chunk(c, b, handle_prev, start_prev_next):
            gather_wait(b)
            add_pos(b, c)
            out_start(c, b)
            if handle_prev:
                pb = (b - 1) % NBUF
                out_wait(pb)
                if start_prev_next:
                    gather_start(c - 1 + NBUF, pb)

        for b in range(NBUF):
            gather_start(b, b)

        for b in range(NBUF):
            do_chunk(b, b, b >= 1, True)

        def group(g, carry):
            c0 = g * NBUF
            for b in range(NBUF):
                do_chunk(c0 + b, b, True, True)
            return carry
        lax.fori_loop(1, nch // NBUF - 1, group, 0)
        for b in range(NBUF):
            c = nch - NBUF + b
            do_chunk(c, b, True, c - 1 + NBUF < nch)
        out_wait((nch - 1) % NBUF)

    return k


def kernel(x, token_table, pos_table):
    B, L = x.shape
    V = token_table.shape[0]
    xf = x.reshape(B * L).astype(jnp.int32)
    tok_p = _build_transpose(V)(token_table.T)
    pos_q = pos_table.reshape(L // 2, 2 * D)
    out = _build_lookup(B * L, V, L)(xf, tok_p, pos_q)
    return out[:, :D].reshape(B, L, D)
